# baseline (speedup 1.0000x reference)
import functools

import jax
import jax.numpy as jnp
from jax import lax
from jax.experimental import pallas as pl
from jax.experimental.pallas import tpu as pltpu
from jax.experimental.pallas import tpu_sc as plsc

BINS = 1024
D = 32
N = 1000000

L = 16
NW = 32
ROWS_PER_TILE = 31248
CHUNK_R = 168
NCHUNK = ROWS_PER_TILE // CHUNK_R
TAIL_R = N - NW * ROWS_PER_TILE
NBUF = 2
ZT_PAD = 1040
QSTRIDE = BINS + 1

_mesh = plsc.VectorSubcoreMesh(core_axis_name="c", subcore_axis_name="s")


@functools.partial(
    pl.kernel,
    out_type=jax.ShapeDtypeStruct((N, D), jnp.float32),
    mesh=_mesh,
    scratch_types=[
        pltpu.VMEM((D * QSTRIDE,), jnp.float32),
        pltpu.VMEM((ZT_PAD,), jnp.float32),
        pltpu.VMEM((CHUNK_R, D), jnp.float32),
        pltpu.VMEM((CHUNK_R, D), jnp.float32),
        pltpu.VMEM((CHUNK_R, D), jnp.float32),
        pltpu.VMEM((CHUNK_R, D), jnp.float32),
        pltpu.SemaphoreType.DMA,
        pltpu.SemaphoreType.DMA,
        pltpu.SemaphoreType.DMA,
        pltpu.SemaphoreType.DMA,
    ],
    compiler_params=pltpu.CompilerParams(needs_layout_passes=False),
)
def _cdf_normalize(x_hbm, qt_hbm, zt_hbm, out_hbm, qt_v, zt_v,
                   xbuf0, xbuf1, obuf0, obuf1,
                   in_sem0, in_sem1, out_sem0, out_sem1):
    xbufs = (xbuf0, xbuf1)
    obufs = (obuf0, obuf1)
    in_sems = (in_sem0, in_sem1)
    out_sems = (out_sem0, out_sem1)
    wid = lax.axis_index("s") * 2 + lax.axis_index("c")
    base = wid * ROWS_PER_TILE

    pltpu.sync_copy(qt_hbm, qt_v)
    pltpu.sync_copy(zt_hbm, zt_v)

    def in_copy(c, b):
        return pltpu.make_async_copy(
            x_hbm.at[pl.ds(base + c * CHUNK_R, CHUNK_R)], xbufs[b], in_sems[b])

    def out_copy(c, b):
        return pltpu.make_async_copy(
            obufs[b], out_hbm.at[pl.ds(base + c * CHUNK_R, CHUNK_R)],
            out_sems[b])

    lane = lax.iota(jnp.int32, L)
    db0 = lane * QSTRIDE - 1
    db1 = db0 + L * QSTRIDE
    piv0 = (plsc.load_gather(qt_v, [db0 + 512]),
            plsc.load_gather(qt_v, [db0 + 1024]))
    piv1 = (plsc.load_gather(qt_v, [db1 + 512]),
            plsc.load_gather(qt_v, [db1 + 1024]))

    def search(xv, db, piv):
        r511, r1023 = piv
        p = jnp.where(r511 < xv, db + 512, db)
        for s in (256, 128, 64, 32, 16, 8, 4, 2, 1):
            pt = p + s
            qv = plsc.load_gather(qt_v, [pt])
            p = jnp.where(qv < xv, pt, p)
        lo = p - db
        idx = lo + jnp.where((lo == BINS - 1) & (r1023 < xv), 1, 0)
        return plsc.load_gather(zt_v, [idx])

    in_copy(0, 0).start()

    @pl.loop(0, NCHUNK, step=NBUF)
    def _chunks(cc):
        for b in range(NBUF):
            c = cc + b
            nb = (b + 1) % NBUF

            @pl.when(c + 1 < NCHUNK)
            def _():
                in_copy(c + 1, nb).start()

            in_copy(c, b).wait()

            @pl.when(c >= NBUF)
            def _():
                out_copy(c - NBUF, b).wait()

            xb = xbufs[b]
            ob = obufs[b]

            @plsc.parallel_loop(0, CHUNK_R, step=1, unroll=8)
            def _row(i):
                ob[i, pl.ds(0, L)] = search(xb[i, pl.ds(0, L)], db0, piv0)
                ob[i, pl.ds(L, L)] = search(xb[i, pl.ds(L, L)], db1, piv1)

            out_copy(c, b).start()

    out_copy(NCHUNK - 2, 0).wait()
    out_copy(NCHUNK - 1, 1).wait()

    @pl.when(wid == NW - 1)
    def _tail():
        t0 = NW * ROWS_PER_TILE
        pltpu.sync_copy(x_hbm.at[pl.ds(t0, TAIL_R)],
                        xbuf0.at[pl.ds(0, TAIL_R)])

        @pl.loop(0, TAIL_R)
        def _row(i):
            obuf0[i, pl.ds(0, L)] = search(xbuf0[i, pl.ds(0, L)], db0, piv0)
            obuf0[i, pl.ds(L, L)] = search(xbuf0[i, pl.ds(L, L)], db1, piv1)

        pltpu.sync_copy(obuf0.at[pl.ds(0, TAIL_R)],
                        out_hbm.at[pl.ds(t0, TAIL_R)])


def kernel(x, quantiles):
    qt = jnp.pad(quantiles.T, ((0, 0), (0, QSTRIDE - BINS)),
                 mode="edge").reshape(-1)
    u = jnp.clip(jnp.arange(ZT_PAD, dtype=jnp.float32) / (BINS - 1),
                 1e-06, 1 - 1e-06)
    zt = jax.scipy.special.erfinv(2.0 * u - 1.0) * 1.41421356
    return _cdf_normalize(x, qt, zt)


# speedup vs baseline: 1.1425x; 1.1425x over previous
import functools

import jax
import jax.numpy as jnp
from jax import lax
from jax.experimental import pallas as pl
from jax.experimental.pallas import tpu as pltpu
from jax.experimental.pallas import tpu_sc as plsc

BINS = 1024
D = 32
N = 1000000
HALF = N // 2

L = 16
NW = 32
ROWS_PER_TILE = 15624
CHUNK_R = 168
NCHUNK = ROWS_PER_TILE // CHUNK_R
TAIL_R = HALF - NW * ROWS_PER_TILE
NBUF = 2
ZT_PAD = 1040
QSTRIDE = BINS + 1

_mesh = plsc.VectorSubcoreMesh(core_axis_name="c", subcore_axis_name="s")


@functools.partial(
    pl.kernel,
    out_type=jax.ShapeDtypeStruct((HALF, D), jnp.float32),
    mesh=_mesh,
    scratch_types=[
        pltpu.VMEM((D * QSTRIDE,), jnp.float32),
        pltpu.VMEM((ZT_PAD,), jnp.float32),
        pltpu.VMEM((CHUNK_R, D), jnp.float32),
        pltpu.VMEM((CHUNK_R, D), jnp.float32),
        pltpu.VMEM((CHUNK_R, D), jnp.float32),
        pltpu.VMEM((CHUNK_R, D), jnp.float32),
        pltpu.SemaphoreType.DMA,
        pltpu.SemaphoreType.DMA,
        pltpu.SemaphoreType.DMA,
        pltpu.SemaphoreType.DMA,
    ],
    compiler_params=pltpu.CompilerParams(needs_layout_passes=False),
)
def _cdf_normalize(x_hbm, qt_hbm, zt_hbm, out_hbm, qt_v, zt_v,
                   xbuf0, xbuf1, obuf0, obuf1,
                   in_sem0, in_sem1, out_sem0, out_sem1):
    xbufs = (xbuf0, xbuf1)
    obufs = (obuf0, obuf1)
    in_sems = (in_sem0, in_sem1)
    out_sems = (out_sem0, out_sem1)
    wid = lax.axis_index("s") * 2 + lax.axis_index("c")
    base = wid * ROWS_PER_TILE

    pltpu.sync_copy(qt_hbm, qt_v)
    pltpu.sync_copy(zt_hbm, zt_v)

    def in_copy(c, b):
        return pltpu.make_async_copy(
            x_hbm.at[pl.ds(base + c * CHUNK_R, CHUNK_R)], xbufs[b], in_sems[b])

    def out_copy(c, b):
        return pltpu.make_async_copy(
            obufs[b], out_hbm.at[pl.ds(base + c * CHUNK_R, CHUNK_R)],
            out_sems[b])

    lane = lax.iota(jnp.int32, L)
    db0 = lane * QSTRIDE - 1
    db1 = db0 + L * QSTRIDE
    piv0 = (plsc.load_gather(qt_v, [db0 + 512]),
            plsc.load_gather(qt_v, [db0 + 1024]))
    piv1 = (plsc.load_gather(qt_v, [db1 + 512]),
            plsc.load_gather(qt_v, [db1 + 1024]))

    def search(xv, db, piv):
        r511, r1023 = piv
        p = jnp.where(r511 < xv, db + 512, db)
        for s in (256, 128, 64, 32, 16, 8, 4, 2, 1):
            pt = p + s
            qv = plsc.load_gather(qt_v, [pt])
            p = jnp.where(qv < xv, pt, p)
        lo = p - db
        idx = lo + jnp.where((lo == BINS - 1) & (r1023 < xv), 1, 0)
        return plsc.load_gather(zt_v, [idx])

    in_copy(0, 0).start()

    @pl.loop(0, NCHUNK - 1, step=NBUF)
    def _chunks(cc):
        for b in range(NBUF):
            c = cc + b
            nb = (b + 1) % NBUF

            @pl.when(c + 1 < NCHUNK)
            def _():
                in_copy(c + 1, nb).start()

            in_copy(c, b).wait()

            @pl.when(c >= NBUF)
            def _():
                out_copy(c - NBUF, b).wait()

            xb = xbufs[b]
            ob = obufs[b]

            @plsc.parallel_loop(0, CHUNK_R, step=1, unroll=4)
            def _row(i):
                ob[i, pl.ds(0, L)] = search(xb[i, pl.ds(0, L)], db0, piv0)
                ob[i, pl.ds(L, L)] = search(xb[i, pl.ds(L, L)], db1, piv1)

            out_copy(c, b).start()

    lc = NCHUNK - 1
    in_copy(lc, 0).wait()
    out_copy(lc - NBUF, 0).wait()

    @plsc.parallel_loop(0, CHUNK_R, step=1, unroll=4)
    def _row_last(i):
        obuf0[i, pl.ds(0, L)] = search(xbuf0[i, pl.ds(0, L)], db0, piv0)
        obuf0[i, pl.ds(L, L)] = search(xbuf0[i, pl.ds(L, L)], db1, piv1)

    out_copy(lc, 0).start()
    out_copy(lc - 1, 1).wait()
    out_copy(lc, 0).wait()

    @pl.when(wid == NW - 1)
    def _tail():
        t0 = NW * ROWS_PER_TILE
        pltpu.sync_copy(x_hbm.at[pl.ds(t0, TAIL_R)],
                        xbuf0.at[pl.ds(0, TAIL_R)])

        @pl.loop(0, TAIL_R)
        def _row(i):
            obuf0[i, pl.ds(0, L)] = search(xbuf0[i, pl.ds(0, L)], db0, piv0)
            obuf0[i, pl.ds(L, L)] = search(xbuf0[i, pl.ds(L, L)], db1, piv1)

        pltpu.sync_copy(obuf0.at[pl.ds(0, TAIL_R)],
                        out_hbm.at[pl.ds(t0, TAIL_R)])


def kernel(x, quantiles):
    qt = jnp.pad(quantiles.T, ((0, 0), (0, QSTRIDE - BINS)),
                 mode="edge").reshape(-1)
    u = jnp.clip(jnp.arange(ZT_PAD, dtype=jnp.float32) / (BINS - 1),
                 1e-06, 1 - 1e-06)
    zt = jax.scipy.special.erfinv(2.0 * u - 1.0) * 1.41421356
    z_lo = _cdf_normalize(x[:HALF], qt, zt)
    z_hi = _cdf_normalize(x[HALF:], qt, zt)
    return jnp.concatenate([z_lo, z_hi], axis=0)
